# baseline (speedup 1.0000x reference)
import functools

import jax
import jax.numpy as jnp
from jax import lax
from jax.experimental import pallas as pl
from jax.experimental.pallas import tpu as pltpu
from jax.experimental.pallas import tpu_sc as plsc

N = 10000
E = 320000
F = 128
H = 256
D = 128
L = 4096

NUM_CORES = 2
NUM_SUBCORES = 16
NW = NUM_CORES * NUM_SUBCORES
LANES = 16

N_PAD = 10240
RB = N_PAD // NW
ASPLIT = 2


def _mesh():
    return plsc.VectorSubcoreMesh(core_axis_name="c", subcore_axis_name="s")


_GATHER_DNUMS = lax.GatherDimensionNumbers(
    offset_dims=(), collapsed_slice_dims=(0,), start_index_map=(0,))


def _lane_gather(x, idx):
    return lax.gather(x, idx[:, None], _GATHER_DNUMS, slice_sizes=(1,),
                      mode=lax.GatherScatterMode.PROMISE_IN_BOUNDS)


def _prefix_sum16(mi):
    iota = lax.iota(jnp.int32, LANES)
    p = mi
    for sh in (1, 2, 4, 8):
        shifted = _lane_gather(p, jnp.maximum(iota - sh, 0))
        p = p + jnp.where(iota >= sh, shifted, 0)
    return p


def _make_segmax(feat, chunk, cap):
    fchunks = feat // LANES
    fw = feat // ASPLIT
    wsteps = fw // LANES
    groups2 = chunk // (2 * LANES)
    nchunks = E // chunk
    npairs = nchunks // 2
    flush_at = cap - 2 * LANES

    @functools.partial(
        pl.kernel,
        out_type=tuple(jax.ShapeDtypeStruct((N_PAD, fw), jnp.float32)
                       for _ in range(ASPLIT)),
        mesh=_mesh(),
        compiler_params=pltpu.CompilerParams(needs_layout_passes=False),
        scratch_types=[
            pltpu.VMEM((chunk,), jnp.int32),
            pltpu.VMEM((chunk,), jnp.int32),
            pltpu.VMEM((chunk,), jnp.int32),
            pltpu.VMEM((chunk,), jnp.int32),
            pltpu.VMEM((cap,), jnp.int32),
            pltpu.VMEM((cap,), jnp.int32),
            [pltpu.VMEM((RB + 1, fw), jnp.float32) for _ in range(ASPLIT)],
            pltpu.VMEM((cap, feat), jnp.float32),
            pltpu.SemaphoreType.DMA,
            pltpu.SemaphoreType.DMA,
            pltpu.SemaphoreType.DMA,
            pltpu.SemaphoreType.DMA,
            pltpu.SemaphoreType.DMA,
        ],
    )
    def seg(msgs_hbm, src_hbm, dst_hbm, out0, out1,
            dstb0, dstb1, srcb0, srcb1, ssrc, sldst, accs, gbuf,
            semd0, semd1, sems0, sems1, semg):
        outs = (out0, out1)
        wid = lax.axis_index("s") * NUM_CORES + lax.axis_index("c")
        base = wid * RB
        dstb = (dstb0, dstb1)
        srcb = (srcb0, srcb1)
        semd = (semd0, semd1)
        sems = (sems0, sems1)

        zero16 = jnp.zeros((LANES,), jnp.float32)

        def zrow(r, _):
            for acc in accs:
                for f in range(wsteps):
                    acc[r, pl.ds(f * LANES, LANES)] = zero16
            return 0

        lax.fori_loop(0, RB + 1, zrow, 0)

        zi16 = jnp.zeros((LANES,), jnp.int32)
        rb16 = jnp.full((LANES,), RB, jnp.int32)
        for g in range(cap // LANES):
            ssrc[pl.ds(g * LANES, LANES)] = zi16
            sldst[pl.ds(g * LANES, LANES)] = rb16

        def fire(b, c):
            off = pl.multiple_of(c * chunk, chunk)
            pltpu.async_copy(dst_hbm.at[pl.ds(off, chunk)], dstb[b], semd[b])
            pltpu.async_copy(src_hbm.at[pl.ds(off, chunk)], srcb[b], sems[b])

        def waitb(b):
            pltpu.make_async_copy(dst_hbm.at[pl.ds(0, chunk)], dstb[b],
                                  semd[b]).wait()
            pltpu.make_async_copy(src_hbm.at[pl.ds(0, chunk)], srcb[b],
                                  sems[b]).wait()

        def accum_batch():
            def accgrp(j, _):
                off = pl.multiple_of(j * LANES, LANES)
                ld16 = sldst[pl.ds(off, LANES)]
                for lane in range(LANES):
                    row = j * LANES + lane
                    r = ld16[lane]
                    for st in range(wsteps):
                        for w in range(ASPLIT):
                            sl = pl.ds(st * LANES, LANES)
                            gsl = pl.ds(w * fw + st * LANES, LANES)
                            accs[w][r, sl] = jnp.maximum(accs[w][r, sl],
                                                         gbuf[row, gsl])
                return 0

            lax.fori_loop(0, cap // LANES, accgrp, 0)

        def flush():
            pltpu.async_copy(msgs_hbm.at[ssrc], gbuf, semg).wait()
            accum_batch()

        def append(sref, goff, ld, m, wptr):
            s = sref[pl.ds(goff, LANES)]
            mi = jnp.where(m, 1, 0).astype(jnp.int32)
            pos = wptr + _prefix_sum16(mi) - 1
            plsc.store_scatter(ssrc, [pos], s, mask=m)
            plsc.store_scatter(sldst, [pos], ld, mask=m)

        def scan_chunk(b, wptr):
            dref = dstb[b]
            sref = srcb[b]
            urb = jnp.uint32(RB)

            def group_body(g, wptr):
                goff0 = pl.multiple_of(g * 2 * LANES, 2 * LANES)
                goff1 = goff0 + LANES
                d0 = dref[pl.ds(goff0, LANES)]
                d1 = dref[pl.ds(goff1, LANES)]
                ld0 = d0 - base
                ld1 = d1 - base
                m0 = plsc.bitcast(ld0, jnp.uint32) < urb
                m1 = plsc.bitcast(ld1, jnp.uint32) < urb
                cnt0 = plsc.all_reduce_population_count(m0)[0]
                cnt1 = plsc.all_reduce_population_count(m1)[0]

                @pl.when(cnt0 > 0)
                def _():
                    append(sref, goff0, ld0, m0, wptr)

                @pl.when(cnt1 > 0)
                def _():
                    append(sref, goff1, ld1, m1, wptr + cnt0)

                wptr2 = wptr + cnt0 + cnt1
                do_flush = wptr2 >= flush_at

                @pl.when(do_flush)
                def _():
                    flush()

                return jnp.where(do_flush, 0, wptr2)

            return lax.fori_loop(0, groups2, group_body, wptr)

        fire(0, 0)

        def pair_body(p, wptr):
            c0 = 2 * p
            fire(1, c0 + 1)
            waitb(0)
            wptr = scan_chunk(0, wptr)

            @pl.when(c0 + 2 < nchunks)
            def _():
                fire(0, c0 + 2)

            waitb(1)
            return scan_chunk(1, wptr)

        wptr = lax.fori_loop(0, npairs, pair_body, jnp.int32(0))

        @pl.when(wptr > 0)
        def _():
            flush()

        for w in range(ASPLIT):
            pltpu.sync_copy(accs[w].at[pl.ds(0, RB)],
                            outs[w].at[pl.ds(base, RB)])

    return seg


_ROWS_PER_W = L // NW


@functools.partial(
    pl.kernel,
    out_type=(
        jax.ShapeDtypeStruct((L, H), jnp.float32),
        jax.ShapeDtypeStruct((L, H), jnp.float32),
    ),
    mesh=_mesh(),
    compiler_params=pltpu.CompilerParams(needs_layout_passes=False),
    scratch_types=[
        pltpu.VMEM((_ROWS_PER_W,), jnp.int32),
        pltpu.VMEM((_ROWS_PER_W, H), jnp.float32),
        pltpu.VMEM((_ROWS_PER_W, H), jnp.float32),
        pltpu.SemaphoreType.DMA,
        pltpu.SemaphoreType.DMA,
    ],
)
def _leaf_gather(h_hbm, agg_hbm, leaf_hbm, outh_hbm, outa_hbm,
                 idx_v, rows_h, rows_a, sem1, sem2):
    wid = lax.axis_index("s") * NUM_CORES + lax.axis_index("c")
    base = wid * _ROWS_PER_W
    pltpu.sync_copy(leaf_hbm.at[pl.ds(base, _ROWS_PER_W)], idx_v)
    cp1 = pltpu.async_copy(h_hbm.at[idx_v], rows_h, sem1)
    cp2 = pltpu.async_copy(agg_hbm.at[idx_v], rows_a, sem2)
    cp1.wait()
    cp2.wait()
    pltpu.sync_copy(rows_h, outh_hbm.at[pl.ds(base, _ROWS_PER_W)])
    pltpu.sync_copy(rows_a, outa_hbm.at[pl.ds(base, _ROWS_PER_W)])


_RBLK = 512


def _mlp1_body(x_ref, wp_ref, bp_ref, o_ref):
    x = x_ref[...]
    o_ref[...] = jax.nn.relu(
        jnp.dot(x, wp_ref[...], preferred_element_type=jnp.float32) + bp_ref[...])


def _tc_mlp1(x, wpT, bp):
    return pl.pallas_call(
        _mlp1_body,
        grid=(N_PAD // _RBLK,),
        in_specs=[
            pl.BlockSpec((_RBLK, F), lambda i: (i, 0)),
            pl.BlockSpec((F, F), lambda i: (0, 0)),
            pl.BlockSpec((1, F), lambda i: (0, 0)),
        ],
        out_specs=pl.BlockSpec((_RBLK, F), lambda i: (i, 0)),
        out_shape=jax.ShapeDtypeStruct((N_PAD, F), jnp.float32),
    )(x, wpT, bp)


def _mid_body(x_ref, agg_ref, ws_ref, wn_ref, bn_ref, wp_ref, bp_ref,
              h_ref, m2_ref):
    h = jnp.tanh(
        jnp.dot(x_ref[...], ws_ref[...], preferred_element_type=jnp.float32)
        + jnp.dot(agg_ref[...], wn_ref[...], preferred_element_type=jnp.float32)
        + bn_ref[...])
    h_ref[...] = h
    m2_ref[...] = jax.nn.relu(
        jnp.dot(h, wp_ref[...], preferred_element_type=jnp.float32) + bp_ref[...])


def _tc_mid(x, agg1, ws1T, wn1T, bn1, wp2T, bp2):
    return pl.pallas_call(
        _mid_body,
        grid=(N_PAD // _RBLK,),
        in_specs=[
            pl.BlockSpec((_RBLK, F), lambda i: (i, 0)),
            pl.BlockSpec((_RBLK, F), lambda i: (i, 0)),
            pl.BlockSpec((F, H), lambda i: (0, 0)),
            pl.BlockSpec((F, H), lambda i: (0, 0)),
            pl.BlockSpec((1, H), lambda i: (0, 0)),
            pl.BlockSpec((H, H), lambda i: (0, 0)),
            pl.BlockSpec((1, H), lambda i: (0, 0)),
        ],
        out_specs=[
            pl.BlockSpec((_RBLK, H), lambda i: (i, 0)),
            pl.BlockSpec((_RBLK, H), lambda i: (i, 0)),
        ],
        out_shape=[
            jax.ShapeDtypeStruct((N_PAD, H), jnp.float32),
            jax.ShapeDtypeStruct((N_PAD, H), jnp.float32),
        ],
    )(x, agg1, ws1T, wn1T, bn1, wp2T, bp2)


def _head_body(hl_ref, al_ref, ws2_ref, wn2_ref, bn2_ref, cmd_ref, wc_ref,
               bc_ref, w3_ref, b3_ref, w4_ref, b4_ref, w5_ref, b5_ref, o_ref):
    emb = (jnp.dot(hl_ref[...], ws2_ref[...], preferred_element_type=jnp.float32)
           + jnp.dot(al_ref[...], wn2_ref[...], preferred_element_type=jnp.float32)
           + bn2_ref[...])
    enc = jnp.dot(cmd_ref[...] * 0.0001, wc_ref[...],
                  preferred_element_type=jnp.float32) + bc_ref[...]
    o = emb * enc
    o = jnp.tanh(jnp.dot(o, w3_ref[...], preferred_element_type=jnp.float32)
                 + b3_ref[...])
    o = jnp.tanh(jnp.dot(o, w4_ref[...], preferred_element_type=jnp.float32)
                 + b4_ref[...])
    o = jnp.tanh(jnp.dot(o, w5_ref[...], preferred_element_type=jnp.float32)
                 + b5_ref[...])
    o_ref[...] = o


def _tc_head(hleaf, aggleaf, ws2T, wn2T, bn2, cmd, wcT, bc,
             w3T, b3, w4T, b4, w5T, b5):
    return pl.pallas_call(
        _head_body,
        out_shape=jax.ShapeDtypeStruct((L, 1), jnp.float32),
    )(hleaf, aggleaf, ws2T, wn2T, bn2, cmd, wcT, bc, w3T, b3, w4T, b4, w5T, b5)


_segmax_f = _make_segmax(F, 3200, 256)
_segmax_h = _make_segmax(H, 1600, 128)


def kernel(node_inputs, edge_index, leaf_nodes, command,
           Wp1, bp1, Ws1, Wn1, bn1, Wp2, bp2, Ws2, Wn2, bn2,
           Wc, bc, W3, b3, W4, b4, W5, b5):
    x = jnp.pad(node_inputs, ((0, N_PAD - N), (0, 0)))
    src = edge_index[0]
    dst = edge_index[1]

    m1 = _tc_mlp1(x, Wp1.T, bp1.reshape(1, F))
    agg1 = jnp.concatenate(_segmax_f(m1, src, dst), axis=1)
    h, m2 = _tc_mid(x, agg1, Ws1.T, Wn1.T, bn1.reshape(1, H),
                    Wp2.T, bp2.reshape(1, H))
    agg2 = jnp.concatenate(_segmax_h(m2, src, dst), axis=1)
    hleaf, aggleaf = _leaf_gather(h, agg2, leaf_nodes)
    o = _tc_head(hleaf, aggleaf, Ws2.T, Wn2.T, bn2.reshape(1, D),
                 command.reshape(1, 2), Wc.T, bc.reshape(1, D),
                 W3.T, b3.reshape(1, 32), W4.T, b4.reshape(1, 32),
                 W5.T, b5.reshape(1, 1))
    return o


# speedup vs baseline: 1.9359x; 1.9359x over previous
import functools

import jax
import jax.numpy as jnp
from jax import lax
from jax.experimental import pallas as pl
from jax.experimental.pallas import tpu as pltpu
from jax.experimental.pallas import tpu_sc as plsc

N = 10000
E = 320000
F = 128
H = 256
D = 128
L = 4096

NUM_CORES = 2
NUM_SUBCORES = 16
NW = NUM_CORES * NUM_SUBCORES
LANES = 16

N_PAD = 10240
RB = N_PAD // NW
ASPLIT = 2


def _mesh():
    return plsc.VectorSubcoreMesh(core_axis_name="c", subcore_axis_name="s")


_GATHER_DNUMS = lax.GatherDimensionNumbers(
    offset_dims=(), collapsed_slice_dims=(0,), start_index_map=(0,))


def _lane_gather(x, idx):
    return lax.gather(x, idx[:, None], _GATHER_DNUMS, slice_sizes=(1,),
                      mode=lax.GatherScatterMode.PROMISE_IN_BOUNDS)


def _prefix_sum16(mi):
    iota = lax.iota(jnp.int32, LANES)
    p = mi
    for sh in (1, 2, 4, 8):
        shifted = _lane_gather(p, jnp.maximum(iota - sh, 0))
        p = p + jnp.where(iota >= sh, shifted, 0)
    return p


def _make_segmax(feat, chunk, cap):
    fchunks = feat // LANES
    fw = feat // ASPLIT
    wsteps = fw // LANES
    groups = chunk // LANES
    nchunks = E // chunk
    npairs = nchunks // 2
    flush_at = cap - LANES

    @functools.partial(
        pl.kernel,
        out_type=tuple(jax.ShapeDtypeStruct((N_PAD, fw), jnp.float32)
                       for _ in range(ASPLIT)),
        mesh=_mesh(),
        compiler_params=pltpu.CompilerParams(needs_layout_passes=False),
        scratch_types=[
            pltpu.VMEM((chunk,), jnp.int32),
            pltpu.VMEM((chunk,), jnp.int32),
            pltpu.VMEM((chunk,), jnp.int32),
            pltpu.VMEM((chunk,), jnp.int32),
            pltpu.VMEM((cap,), jnp.int32),
            pltpu.VMEM((cap,), jnp.int32),
            pltpu.VMEM((cap,), jnp.int32),
            pltpu.VMEM((cap,), jnp.int32),
            [pltpu.VMEM((RB + 1, fw), jnp.float32) for _ in range(ASPLIT)],
            pltpu.VMEM((cap, feat), jnp.float32),
            pltpu.SemaphoreType.DMA,
            pltpu.SemaphoreType.DMA,
            pltpu.SemaphoreType.DMA,
            pltpu.SemaphoreType.DMA,
            pltpu.SemaphoreType.DMA,
        ],
    )
    def seg(msgs_hbm, src_hbm, dst_hbm, out0, out1,
            dstb0, dstb1, srcb0, srcb1, ssrc, sldst, midx, mhold, accs, gbuf,
            semd0, semd1, sems0, sems1, semg):
        outs = (out0, out1)
        wid = lax.axis_index("s") * NUM_CORES + lax.axis_index("c")
        base = wid * RB
        dstb = (dstb0, dstb1)
        srcb = (srcb0, srcb1)
        semd = (semd0, semd1)
        sems = (sems0, sems1)

        zero16 = jnp.zeros((LANES,), jnp.float32)

        def zrow(r, _):
            for acc in accs:
                for f in range(wsteps):
                    acc[r, pl.ds(f * LANES, LANES)] = zero16
            return 0

        lax.fori_loop(0, RB + 1, zrow, 0)

        zi16 = jnp.zeros((LANES,), jnp.int32)
        rb16 = jnp.full((LANES,), RB, jnp.int32)
        for g in range(cap // LANES):
            ssrc[pl.ds(g * LANES, LANES)] = zi16
            sldst[pl.ds(g * LANES, LANES)] = rb16

        def fire(b, c):
            off = pl.multiple_of(c * chunk, chunk)
            pltpu.async_copy(dst_hbm.at[pl.ds(off, chunk)], dstb[b], semd[b])
            pltpu.async_copy(src_hbm.at[pl.ds(off, chunk)], srcb[b], sems[b])

        def waitb(b):
            pltpu.make_async_copy(dst_hbm.at[pl.ds(0, chunk)], dstb[b],
                                  semd[b]).wait()
            pltpu.make_async_copy(src_hbm.at[pl.ds(0, chunk)], srcb[b],
                                  sems[b]).wait()

        def accum_batch():
            def accgrp(j, _):
                off = pl.multiple_of(j * LANES, LANES)
                ld16 = mhold[pl.ds(off, LANES)]
                for lane in range(LANES):
                    row = j * LANES + lane
                    r = ld16[lane]
                    for st in range(wsteps):
                        for w in range(ASPLIT):
                            sl = pl.ds(st * LANES, LANES)
                            gsl = pl.ds(w * fw + st * LANES, LANES)
                            accs[w][r, sl] = jnp.maximum(accs[w][r, sl],
                                                         gbuf[row, gsl])
                return 0

            lax.fori_loop(0, cap // LANES, accgrp, 0)

        def wait_gather():
            pltpu.make_async_copy(msgs_hbm.at[midx], gbuf, semg).wait()

        def flush(pending):
            @pl.when(pending > 0)
            def _():
                wait_gather()
                accum_batch()

            for g in range(cap // LANES):
                sl = pl.ds(g * LANES, LANES)
                midx[sl] = ssrc[sl]
                mhold[sl] = sldst[sl]
            pltpu.async_copy(msgs_hbm.at[midx], gbuf, semg)

        def append(sref, goff, ld, m, wptr):
            s = sref[pl.ds(goff, LANES)]
            mi = jnp.where(m, 1, 0).astype(jnp.int32)
            pos = wptr + _prefix_sum16(mi) - 1
            plsc.store_scatter(ssrc, [pos], s, mask=m)
            plsc.store_scatter(sldst, [pos], ld, mask=m)

        def scan_chunk(b, carry):
            dref = dstb[b]
            sref = srcb[b]
            urb = jnp.uint32(RB)

            def group_body(g, carry):
                wptr, pending = carry
                goff = pl.multiple_of(g * LANES, LANES)
                d = dref[pl.ds(goff, LANES)]
                ld = d - base
                m = plsc.bitcast(ld, jnp.uint32) < urb
                cnt = plsc.all_reduce_population_count(m)[0]

                @pl.when(cnt > 0)
                def _():
                    append(sref, goff, ld, m, wptr)

                wptr2 = wptr + cnt
                do_flush = wptr2 >= flush_at

                @pl.when(do_flush)
                def _():
                    flush(pending)

                return (jnp.where(do_flush, 0, wptr2),
                        jnp.where(do_flush, 1, pending))

            return lax.fori_loop(0, groups, group_body, carry)

        fire(0, 0)

        def pair_body(p, carry):
            c0 = 2 * p
            fire(1, c0 + 1)
            waitb(0)
            carry = scan_chunk(0, carry)

            @pl.when(c0 + 2 < nchunks)
            def _():
                fire(0, c0 + 2)

            waitb(1)
            return scan_chunk(1, carry)

        wptr, pending = lax.fori_loop(0, npairs, pair_body,
                                      (jnp.int32(0), jnp.int32(0)))

        @pl.when(pending > 0)
        def _():
            wait_gather()
            accum_batch()

        @pl.when(wptr > 0)
        def _():
            flush(jnp.int32(0))
            wait_gather()
            accum_batch()

        for w in range(ASPLIT):
            pltpu.sync_copy(accs[w].at[pl.ds(0, RB)],
                            outs[w].at[pl.ds(base, RB)])

    return seg


_ROWS_PER_W = L // NW


@functools.partial(
    pl.kernel,
    out_type=(
        jax.ShapeDtypeStruct((L, H), jnp.float32),
        jax.ShapeDtypeStruct((L, H), jnp.float32),
    ),
    mesh=_mesh(),
    compiler_params=pltpu.CompilerParams(needs_layout_passes=False),
    scratch_types=[
        pltpu.VMEM((_ROWS_PER_W,), jnp.int32),
        pltpu.VMEM((_ROWS_PER_W, H), jnp.float32),
        pltpu.VMEM((_ROWS_PER_W, H), jnp.float32),
        pltpu.SemaphoreType.DMA,
        pltpu.SemaphoreType.DMA,
    ],
)
def _leaf_gather(h_hbm, agg_hbm, leaf_hbm, outh_hbm, outa_hbm,
                 idx_v, rows_h, rows_a, sem1, sem2):
    wid = lax.axis_index("s") * NUM_CORES + lax.axis_index("c")
    base = wid * _ROWS_PER_W
    pltpu.sync_copy(leaf_hbm.at[pl.ds(base, _ROWS_PER_W)], idx_v)
    cp1 = pltpu.async_copy(h_hbm.at[idx_v], rows_h, sem1)
    cp2 = pltpu.async_copy(agg_hbm.at[idx_v], rows_a, sem2)
    cp1.wait()
    cp2.wait()
    pltpu.sync_copy(rows_h, outh_hbm.at[pl.ds(base, _ROWS_PER_W)])
    pltpu.sync_copy(rows_a, outa_hbm.at[pl.ds(base, _ROWS_PER_W)])


_RBLK = 512


def _mlp1_body(x_ref, wp_ref, bp_ref, o_ref):
    x = x_ref[...]
    o_ref[...] = jax.nn.relu(
        jnp.dot(x, wp_ref[...], preferred_element_type=jnp.float32) + bp_ref[...])


def _tc_mlp1(x, wpT, bp):
    return pl.pallas_call(
        _mlp1_body,
        grid=(N_PAD // _RBLK,),
        in_specs=[
            pl.BlockSpec((_RBLK, F), lambda i: (i, 0)),
            pl.BlockSpec((F, F), lambda i: (0, 0)),
            pl.BlockSpec((1, F), lambda i: (0, 0)),
        ],
        out_specs=pl.BlockSpec((_RBLK, F), lambda i: (i, 0)),
        out_shape=jax.ShapeDtypeStruct((N_PAD, F), jnp.float32),
    )(x, wpT, bp)


def _mid_body(x_ref, agg_ref, ws_ref, wn_ref, bn_ref, wp_ref, bp_ref,
              h_ref, m2_ref):
    h = jnp.tanh(
        jnp.dot(x_ref[...], ws_ref[...], preferred_element_type=jnp.float32)
        + jnp.dot(agg_ref[...], wn_ref[...], preferred_element_type=jnp.float32)
        + bn_ref[...])
    h_ref[...] = h
    m2_ref[...] = jax.nn.relu(
        jnp.dot(h, wp_ref[...], preferred_element_type=jnp.float32) + bp_ref[...])


def _tc_mid(x, agg1, ws1T, wn1T, bn1, wp2T, bp2):
    return pl.pallas_call(
        _mid_body,
        grid=(N_PAD // _RBLK,),
        in_specs=[
            pl.BlockSpec((_RBLK, F), lambda i: (i, 0)),
            pl.BlockSpec((_RBLK, F), lambda i: (i, 0)),
            pl.BlockSpec((F, H), lambda i: (0, 0)),
            pl.BlockSpec((F, H), lambda i: (0, 0)),
            pl.BlockSpec((1, H), lambda i: (0, 0)),
            pl.BlockSpec((H, H), lambda i: (0, 0)),
            pl.BlockSpec((1, H), lambda i: (0, 0)),
        ],
        out_specs=[
            pl.BlockSpec((_RBLK, H), lambda i: (i, 0)),
            pl.BlockSpec((_RBLK, H), lambda i: (i, 0)),
        ],
        out_shape=[
            jax.ShapeDtypeStruct((N_PAD, H), jnp.float32),
            jax.ShapeDtypeStruct((N_PAD, H), jnp.float32),
        ],
    )(x, agg1, ws1T, wn1T, bn1, wp2T, bp2)


def _head_body(hl_ref, al_ref, ws2_ref, wn2_ref, bn2_ref, cmd_ref, wc_ref,
               bc_ref, w3_ref, b3_ref, w4_ref, b4_ref, w5_ref, b5_ref, o_ref):
    emb = (jnp.dot(hl_ref[...], ws2_ref[...], preferred_element_type=jnp.float32)
           + jnp.dot(al_ref[...], wn2_ref[...], preferred_element_type=jnp.float32)
           + bn2_ref[...])
    enc = jnp.dot(cmd_ref[...] * 0.0001, wc_ref[...],
                  preferred_element_type=jnp.float32) + bc_ref[...]
    o = emb * enc
    o = jnp.tanh(jnp.dot(o, w3_ref[...], preferred_element_type=jnp.float32)
                 + b3_ref[...])
    o = jnp.tanh(jnp.dot(o, w4_ref[...], preferred_element_type=jnp.float32)
                 + b4_ref[...])
    o = jnp.tanh(jnp.dot(o, w5_ref[...], preferred_element_type=jnp.float32)
                 + b5_ref[...])
    o_ref[...] = o


def _tc_head(hleaf, aggleaf, ws2T, wn2T, bn2, cmd, wcT, bc,
             w3T, b3, w4T, b4, w5T, b5):
    return pl.pallas_call(
        _head_body,
        out_shape=jax.ShapeDtypeStruct((L, 1), jnp.float32),
    )(hleaf, aggleaf, ws2T, wn2T, bn2, cmd, wcT, bc, w3T, b3, w4T, b4, w5T, b5)


_segmax_f = _make_segmax(F, 3200, 256)
_segmax_h = _make_segmax(H, 1600, 128)


def kernel(node_inputs, edge_index, leaf_nodes, command,
           Wp1, bp1, Ws1, Wn1, bn1, Wp2, bp2, Ws2, Wn2, bn2,
           Wc, bc, W3, b3, W4, b4, W5, b5):
    x = jnp.pad(node_inputs, ((0, N_PAD - N), (0, 0)))
    src = edge_index[0]
    dst = edge_index[1]

    m1 = _tc_mlp1(x, Wp1.T, bp1.reshape(1, F))
    agg1 = jnp.concatenate(_segmax_f(m1, src, dst), axis=1)
    h, m2 = _tc_mid(x, agg1, Ws1.T, Wn1.T, bn1.reshape(1, H),
                    Wp2.T, bp2.reshape(1, H))
    agg2 = jnp.concatenate(_segmax_h(m2, src, dst), axis=1)
    hleaf, aggleaf = _leaf_gather(h, agg2, leaf_nodes)
    o = _tc_head(hleaf, aggleaf, Ws2.T, Wn2.T, bn2.reshape(1, D),
                 command.reshape(1, 2), Wc.T, bc.reshape(1, D),
                 W3.T, b3.reshape(1, 32), W4.T, b4.reshape(1, 32),
                 W5.T, b5.reshape(1, 1))
    return o


# speedup vs baseline: 2.0647x; 1.0665x over previous
import functools

import jax
import jax.numpy as jnp
from jax import lax
from jax.experimental import pallas as pl
from jax.experimental.pallas import tpu as pltpu
from jax.experimental.pallas import tpu_sc as plsc

N = 10000
E = 320000
F = 128
H = 256
D = 128
L = 4096

NUM_CORES = 2
NUM_SUBCORES = 16
NW = NUM_CORES * NUM_SUBCORES
LANES = 16

N_PAD = 10240
RB = N_PAD // NW
ASPLIT = 2


def _mesh():
    return plsc.VectorSubcoreMesh(core_axis_name="c", subcore_axis_name="s")


_GATHER_DNUMS = lax.GatherDimensionNumbers(
    offset_dims=(), collapsed_slice_dims=(0,), start_index_map=(0,))


def _lane_gather(x, idx):
    return lax.gather(x, idx[:, None], _GATHER_DNUMS, slice_sizes=(1,),
                      mode=lax.GatherScatterMode.PROMISE_IN_BOUNDS)


def _prefix_sum16(mi):
    iota = lax.iota(jnp.int32, LANES)
    p = mi
    for sh in (1, 2, 4, 8):
        shifted = _lane_gather(p, jnp.maximum(iota - sh, 0))
        p = p + jnp.where(iota >= sh, shifted, 0)
    return p


def _make_segmax(feat, chunk, cap):
    fchunks = feat // LANES
    fw = feat // ASPLIT
    wsteps = fw // LANES
    groups = chunk // LANES
    nchunks = E // chunk
    npairs = nchunks // 2
    flush_at = cap - LANES

    @functools.partial(
        pl.kernel,
        out_type=tuple(jax.ShapeDtypeStruct((N_PAD, fw), jnp.float32)
                       for _ in range(ASPLIT)),
        mesh=_mesh(),
        compiler_params=pltpu.CompilerParams(needs_layout_passes=False),
        scratch_types=[
            pltpu.VMEM((chunk,), jnp.int32),
            pltpu.VMEM((chunk,), jnp.int32),
            pltpu.VMEM((chunk,), jnp.int32),
            pltpu.VMEM((chunk,), jnp.int32),
            pltpu.VMEM((cap,), jnp.int32),
            pltpu.VMEM((cap,), jnp.int32),
            pltpu.VMEM((cap,), jnp.int32),
            pltpu.VMEM((cap,), jnp.int32),
            [pltpu.VMEM((RB + 1, fw), jnp.float32) for _ in range(ASPLIT)],
            pltpu.VMEM((cap, feat), jnp.float32),
            pltpu.SemaphoreType.DMA,
            pltpu.SemaphoreType.DMA,
            pltpu.SemaphoreType.DMA,
            pltpu.SemaphoreType.DMA,
            pltpu.SemaphoreType.DMA,
        ],
    )
    def seg(msgs_hbm, src_hbm, dst_hbm, out0, out1,
            dstb0, dstb1, srcb0, srcb1, ssrc, sldst, midx, mhold, accs, gbuf,
            semd0, semd1, sems0, sems1, semg):
        outs = (out0, out1)
        wid = lax.axis_index("s") * NUM_CORES + lax.axis_index("c")
        base = wid * RB
        dstb = (dstb0, dstb1)
        srcb = (srcb0, srcb1)
        semd = (semd0, semd1)
        sems = (sems0, sems1)

        zero16 = jnp.zeros((LANES,), jnp.float32)

        def zrow(r, _):
            for acc in accs:
                for f in range(wsteps):
                    acc[r, pl.ds(f * LANES, LANES)] = zero16
            return 0

        lax.fori_loop(0, RB + 1, zrow, 0)

        zi16 = jnp.zeros((LANES,), jnp.int32)
        rb16 = jnp.full((LANES,), RB, jnp.int32)
        for g in range(cap // LANES):
            ssrc[pl.ds(g * LANES, LANES)] = zi16
            sldst[pl.ds(g * LANES, LANES)] = rb16

        def fire(b, c):
            off = pl.multiple_of(c * chunk, chunk)
            pltpu.async_copy(dst_hbm.at[pl.ds(off, chunk)], dstb[b], semd[b])
            pltpu.async_copy(src_hbm.at[pl.ds(off, chunk)], srcb[b], sems[b])

        def waitb(b):
            pltpu.make_async_copy(dst_hbm.at[pl.ds(0, chunk)], dstb[b],
                                  semd[b]).wait()
            pltpu.make_async_copy(src_hbm.at[pl.ds(0, chunk)], srcb[b],
                                  sems[b]).wait()

        def accum_batch():
            def accgrp(j, _):
                off = pl.multiple_of(j * LANES, LANES)
                ld16 = mhold[pl.ds(off, LANES)]
                for lane in range(LANES):
                    row = j * LANES + lane
                    r = ld16[lane]
                    for st in range(wsteps):
                        for w in range(ASPLIT):
                            sl = pl.ds(st * LANES, LANES)
                            gsl = pl.ds(w * fw + st * LANES, LANES)
                            accs[w][r, sl] = jnp.maximum(accs[w][r, sl],
                                                         gbuf[row, gsl])
                return 0

            lax.fori_loop(0, cap // LANES, accgrp, 0)

        def wait_gather():
            pltpu.make_async_copy(msgs_hbm.at[midx], gbuf, semg).wait()

        def flush(pending):
            @pl.when(pending > 0)
            def _():
                wait_gather()
                accum_batch()

            for g in range(cap // LANES):
                sl = pl.ds(g * LANES, LANES)
                midx[sl] = ssrc[sl]
                mhold[sl] = sldst[sl]
            pltpu.async_copy(msgs_hbm.at[midx], gbuf, semg)

        def append(sref, goff, ld, m, wptr):
            s = sref[pl.ds(goff, LANES)]
            mi = jnp.where(m, 1, 0).astype(jnp.int32)
            pos = wptr + _prefix_sum16(mi) - 1
            plsc.store_scatter(ssrc, [pos], s, mask=m)
            plsc.store_scatter(sldst, [pos], ld, mask=m)

        def scan_chunk(b, carry):
            dref = dstb[b]
            sref = srcb[b]
            urb = jnp.uint32(RB)

            def group_body(g, carry):
                wptr, pending = carry
                goff = pl.multiple_of(g * LANES, LANES)
                d = dref[pl.ds(goff, LANES)]
                ld = d - base
                m = plsc.bitcast(ld, jnp.uint32) < urb
                cnt = plsc.all_reduce_population_count(m)[0]

                @pl.when(cnt > 0)
                def _():
                    append(sref, goff, ld, m, wptr)

                wptr2 = wptr + cnt
                do_flush = wptr2 >= flush_at

                @pl.when(do_flush)
                def _():
                    flush(pending)

                return (jnp.where(do_flush, 0, wptr2),
                        jnp.where(do_flush, 1, pending))

            return lax.fori_loop(0, groups, group_body, carry)

        fire(0, 0)

        def pair_body(p, carry):
            c0 = 2 * p
            fire(1, c0 + 1)
            waitb(0)
            carry = scan_chunk(0, carry)

            @pl.when(c0 + 2 < nchunks)
            def _():
                fire(0, c0 + 2)

            waitb(1)
            return scan_chunk(1, carry)

        wptr, pending = lax.fori_loop(0, npairs, pair_body,
                                      (jnp.int32(0), jnp.int32(0)))

        @pl.when(pending > 0)
        def _():
            wait_gather()
            accum_batch()

        @pl.when(wptr > 0)
        def _():
            flush(jnp.int32(0))
            wait_gather()
            accum_batch()

        for w in range(ASPLIT):
            pltpu.sync_copy(accs[w].at[pl.ds(0, RB)],
                            outs[w].at[pl.ds(base, RB)])

    return seg


_ROWS_PER_W = L // NW


@functools.partial(
    pl.kernel,
    out_type=(
        jax.ShapeDtypeStruct((L, H), jnp.float32),
        jax.ShapeDtypeStruct((L, H), jnp.float32),
    ),
    mesh=_mesh(),
    compiler_params=pltpu.CompilerParams(needs_layout_passes=False),
    scratch_types=[
        pltpu.VMEM((_ROWS_PER_W,), jnp.int32),
        pltpu.VMEM((_ROWS_PER_W, H), jnp.float32),
        pltpu.VMEM((_ROWS_PER_W, H), jnp.float32),
        pltpu.SemaphoreType.DMA,
        pltpu.SemaphoreType.DMA,
    ],
)
def _leaf_gather(h_hbm, agg_hbm, leaf_hbm, outh_hbm, outa_hbm,
                 idx_v, rows_h, rows_a, sem1, sem2):
    wid = lax.axis_index("s") * NUM_CORES + lax.axis_index("c")
    base = wid * _ROWS_PER_W
    pltpu.sync_copy(leaf_hbm.at[pl.ds(base, _ROWS_PER_W)], idx_v)
    cp1 = pltpu.async_copy(h_hbm.at[idx_v], rows_h, sem1)
    cp2 = pltpu.async_copy(agg_hbm.at[idx_v], rows_a, sem2)
    cp1.wait()
    cp2.wait()
    pltpu.sync_copy(rows_h, outh_hbm.at[pl.ds(base, _ROWS_PER_W)])
    pltpu.sync_copy(rows_a, outa_hbm.at[pl.ds(base, _ROWS_PER_W)])


_RBLK = 512


def _mlp1_body(x_ref, wp_ref, bp_ref, o_ref):
    x = x_ref[...]
    o_ref[...] = jax.nn.relu(
        jnp.dot(x, wp_ref[...], preferred_element_type=jnp.float32) + bp_ref[...])


def _tc_mlp1(x, wpT, bp):
    return pl.pallas_call(
        _mlp1_body,
        grid=(N_PAD // _RBLK,),
        in_specs=[
            pl.BlockSpec((_RBLK, F), lambda i: (i, 0)),
            pl.BlockSpec((F, F), lambda i: (0, 0)),
            pl.BlockSpec((1, F), lambda i: (0, 0)),
        ],
        out_specs=pl.BlockSpec((_RBLK, F), lambda i: (i, 0)),
        out_shape=jax.ShapeDtypeStruct((N_PAD, F), jnp.float32),
    )(x, wpT, bp)


def _mid_body(x_ref, agg_ref, ws_ref, wn_ref, bn_ref, wp_ref, bp_ref,
              h_ref, m2_ref):
    h = jnp.tanh(
        jnp.dot(x_ref[...], ws_ref[...], preferred_element_type=jnp.float32)
        + jnp.dot(agg_ref[...], wn_ref[...], preferred_element_type=jnp.float32)
        + bn_ref[...])
    h_ref[...] = h
    m2_ref[...] = jax.nn.relu(
        jnp.dot(h, wp_ref[...], preferred_element_type=jnp.float32) + bp_ref[...])


def _tc_mid(x, agg1, ws1T, wn1T, bn1, wp2T, bp2):
    return pl.pallas_call(
        _mid_body,
        grid=(N_PAD // _RBLK,),
        in_specs=[
            pl.BlockSpec((_RBLK, F), lambda i: (i, 0)),
            pl.BlockSpec((_RBLK, F), lambda i: (i, 0)),
            pl.BlockSpec((F, H), lambda i: (0, 0)),
            pl.BlockSpec((F, H), lambda i: (0, 0)),
            pl.BlockSpec((1, H), lambda i: (0, 0)),
            pl.BlockSpec((H, H), lambda i: (0, 0)),
            pl.BlockSpec((1, H), lambda i: (0, 0)),
        ],
        out_specs=[
            pl.BlockSpec((_RBLK, H), lambda i: (i, 0)),
            pl.BlockSpec((_RBLK, H), lambda i: (i, 0)),
        ],
        out_shape=[
            jax.ShapeDtypeStruct((N_PAD, H), jnp.float32),
            jax.ShapeDtypeStruct((N_PAD, H), jnp.float32),
        ],
    )(x, agg1, ws1T, wn1T, bn1, wp2T, bp2)


def _head_body(hl_ref, al_ref, ws2_ref, wn2_ref, bn2_ref, cmd_ref, wc_ref,
               bc_ref, w3_ref, b3_ref, w4_ref, b4_ref, w5_ref, b5_ref, o_ref):
    emb = (jnp.dot(hl_ref[...], ws2_ref[...], preferred_element_type=jnp.float32)
           + jnp.dot(al_ref[...], wn2_ref[...], preferred_element_type=jnp.float32)
           + bn2_ref[...])
    enc = jnp.dot(cmd_ref[...] * 0.0001, wc_ref[...],
                  preferred_element_type=jnp.float32) + bc_ref[...]
    o = emb * enc
    o = jnp.tanh(jnp.dot(o, w3_ref[...], preferred_element_type=jnp.float32)
                 + b3_ref[...])
    o = jnp.tanh(jnp.dot(o, w4_ref[...], preferred_element_type=jnp.float32)
                 + b4_ref[...])
    o = jnp.tanh(jnp.dot(o, w5_ref[...], preferred_element_type=jnp.float32)
                 + b5_ref[...])
    o_ref[...] = o


def _tc_head(hleaf, aggleaf, ws2T, wn2T, bn2, cmd, wcT, bc,
             w3T, b3, w4T, b4, w5T, b5):
    return pl.pallas_call(
        _head_body,
        out_shape=jax.ShapeDtypeStruct((L, 1), jnp.float32),
    )(hleaf, aggleaf, ws2T, wn2T, bn2, cmd, wcT, bc, w3T, b3, w4T, b4, w5T, b5)


_segmax_f = _make_segmax(F, 3200, 256)
_segmax_h = _make_segmax(H, 800, 160)


def kernel(node_inputs, edge_index, leaf_nodes, command,
           Wp1, bp1, Ws1, Wn1, bn1, Wp2, bp2, Ws2, Wn2, bn2,
           Wc, bc, W3, b3, W4, b4, W5, b5):
    x = jnp.pad(node_inputs, ((0, N_PAD - N), (0, 0)))
    src = edge_index[0]
    dst = edge_index[1]

    m1 = _tc_mlp1(x, Wp1.T, bp1.reshape(1, F))
    agg1 = jnp.concatenate(_segmax_f(m1, src, dst), axis=1)
    h, m2 = _tc_mid(x, agg1, Ws1.T, Wn1.T, bn1.reshape(1, H),
                    Wp2.T, bp2.reshape(1, H))
    agg2 = jnp.concatenate(_segmax_h(m2, src, dst), axis=1)
    hleaf, aggleaf = _leaf_gather(h, agg2, leaf_nodes)
    o = _tc_head(hleaf, aggleaf, Ws2.T, Wn2.T, bn2.reshape(1, D),
                 command.reshape(1, 2), Wc.T, bc.reshape(1, D),
                 W3.T, b3.reshape(1, 32), W4.T, b4.reshape(1, 32),
                 W5.T, b5.reshape(1, 1))
    return o
